# baseline (speedup 1.0000x reference)
import functools

import jax
import jax.numpy as jnp
from jax import lax
from jax.experimental import pallas as pl
from jax.experimental.pallas import tpu as pltpu
from jax.experimental.pallas import tpu_sc as plsc

NB = 8
N = 100000
G = 512
NW = 32
CHUNK = 80
NCHUNKS = N // CHUNK
CPW = (NCHUNKS + NW - 1) // NW
GV = G // 16


def _worker_id():
    return lax.axis_index("s") * 2 + lax.axis_index("c")


def _sc_stats_body(x_hbm, lab_hbm, seg_out, ssq_out, xbuf, labbuf, accbuf,
                   ssqbuf):
    wid = _worker_id()
    zero16 = jnp.zeros((16,), jnp.float32)
    for j in range(NB * GV):
        accbuf[pl.ds(16 * j, 16)] = zero16
    for j in range(GV):
        ssqbuf[pl.ds(16 * j, 16)] = zero16

    def chunk_body(i, _):
        c = wid + NW * i

        @pl.when(c < NCHUNKS)
        def _():
            row0 = c * CHUNK
            pltpu.sync_copy(x_hbm.at[pl.ds(row0, CHUNK)], xbuf)
            pltpu.sync_copy(lab_hbm.at[pl.ds(row0, CHUNK)], labbuf)
            def group_body(g, accs):
                labv = labbuf[pl.ds(g * 16, 16)]
                new_accs = list(accs)
                for k in range(16):
                    rr = g * 16 + k
                    off = labv[k] * G
                    for j in range(GV):
                        x = xbuf[rr, pl.ds(16 * j, 16)]
                        plsc.addupdate(
                            accbuf.at[pl.ds(off + 16 * j, 16)], x)
                        new_accs[j] = new_accs[j] + x * x
                return tuple(new_accs)

            accs = lax.fori_loop(0, CHUNK // 16, group_body,
                                 (zero16,) * GV)
            for j in range(GV):
                s = ssqbuf[pl.ds(16 * j, 16)]
                ssqbuf[pl.ds(16 * j, 16)] = s + accs[j]

        return 0

    lax.fori_loop(0, CPW, chunk_body, 0)
    pltpu.sync_copy(accbuf, seg_out.at[wid])
    pltpu.sync_copy(ssqbuf, ssq_out.at[wid])


def _sc_apply_body(x_hbm, lab_hbm, tbl_hbm, out_hbm, xbuf, labbuf, corrbuf,
                   sem):
    wid = _worker_id()

    def chunk_body(i, _):
        c = wid + NW * i

        @pl.when(c < NCHUNKS)
        def _():
            row0 = c * CHUNK
            pltpu.sync_copy(x_hbm.at[pl.ds(row0, CHUNK)], xbuf)
            pltpu.sync_copy(lab_hbm.at[pl.ds(row0, CHUNK)], labbuf)
            pltpu.async_copy(tbl_hbm.at[labbuf], corrbuf, sem).wait()
            for j in range(GV):
                def row_body(r, acc):
                    v = corrbuf[r, pl.ds(16 * j, 16)]
                    plsc.addupdate(xbuf.at[r, pl.ds(16 * j, 16)], v)
                    return acc
                lax.fori_loop(0, CHUNK, row_body, 0)
            pltpu.sync_copy(xbuf, out_hbm.at[pl.ds(row0, CHUNK)])

        return 0

    lax.fori_loop(0, CPW, chunk_body, 0)


def _finalize_body(segp_ref, ssqp_ref, lab_ref, tbl_ref):
    seg = jnp.sum(segp_ref[...], axis=0)
    ssq = jnp.sum(ssqp_ref[...], axis=0, keepdims=True)
    gm = jnp.sum(seg, axis=0, keepdims=True) / N
    gv = ssq / N - gm * gm
    inv_std = 1.0 / (jnp.sqrt(gv) + 1e-8)
    labs = lab_ref[:, 0, :]
    rows = []
    for b in range(NB):
        cnt = jnp.maximum(
            jnp.sum((labs == b).astype(jnp.float32)), 1.0)
        bm = seg[b:b + 1, :] / cnt
        rows.append(-(bm - gm) * inv_std)
    tbl_ref[...] = jnp.concatenate(rows, axis=0)


@jax.jit
def kernel(expression, batch_labels):
    mesh = plsc.VectorSubcoreMesh(core_axis_name="c", subcore_axis_name="s")

    stats = functools.partial(
        pl.kernel,
        mesh=mesh,
        out_type=[
            jax.ShapeDtypeStruct((NW, NB * G), jnp.float32),
            jax.ShapeDtypeStruct((NW, G), jnp.float32),
        ],
        scratch_types=[
            pltpu.VMEM((CHUNK, G), jnp.float32),
            pltpu.VMEM((CHUNK,), jnp.int32),
            pltpu.VMEM((NB * G,), jnp.float32),
            pltpu.VMEM((G,), jnp.float32),
        ],
    )(_sc_stats_body)
    seg_p, ssq_p = stats(expression, batch_labels)
    seg_p = seg_p.reshape(NW, NB, G)

    labels3 = batch_labels.reshape(50, 1, N // 50)
    negtbl = pl.pallas_call(
        _finalize_body,
        out_shape=jax.ShapeDtypeStruct((NB, G), jnp.float32),
    )(seg_p, ssq_p, labels3)

    apply_fn = functools.partial(
        pl.kernel,
        mesh=mesh,
        out_type=jax.ShapeDtypeStruct((N, G), jnp.float32),
        scratch_types=[
            pltpu.VMEM((CHUNK, G), jnp.float32),
            pltpu.VMEM((CHUNK,), jnp.int32),
            pltpu.VMEM((CHUNK, G), jnp.float32),
            pltpu.SemaphoreType.DMA,
        ],
    )(_sc_apply_body)
    return apply_fn(expression, batch_labels, negtbl)


# speedup vs baseline: 1.8683x; 1.8683x over previous
import functools

import jax
import jax.numpy as jnp
from jax import lax
from jax.experimental import pallas as pl
from jax.experimental.pallas import tpu as pltpu
from jax.experimental.pallas import tpu_sc as plsc

NB = 8
N = 100000
G = 512
NW = 32
CHUNK = 80
NCHUNKS = N // CHUNK
CPW = NCHUNKS // NW
EXTRA = NCHUNKS % NW
MAXC = CPW + 1
GV = G // 16
RG = CHUNK // 16


def _worker_id():
    return lax.axis_index("s") * 2 + lax.axis_index("c")


def _my_chunks(wid):
    start = wid * CPW + jnp.minimum(wid, EXTRA)
    cnt = jnp.where(wid < EXTRA, CPW + 1, CPW)
    return start, cnt


def _tree_sum(vals):
    vals = list(vals)
    while len(vals) > 1:
        vals = [vals[i] + vals[i + 1] for i in range(0, len(vals) - 1, 2)] + (
            [vals[-1]] if len(vals) % 2 else [])
    return vals[0]


def _sc_stats_body(x_hbm, lab_hbm, seg_out, ssq_out, xbuf, laball, accbuf,
                   ssqbuf):
    wid = _worker_id()
    start, cnt = _my_chunks(wid)
    zero16 = jnp.zeros((16,), jnp.float32)
    for j in range(NB * GV):
        accbuf[pl.ds(16 * j, 16)] = zero16
    for j in range(GV):
        ssqbuf[pl.ds(16 * j, 16)] = zero16
    pltpu.sync_copy(lab_hbm.at[pl.ds(start * CHUNK, CPW * CHUNK)], laball.at[pl.ds(0, CPW * CHUNK)])

    @pl.when(cnt > CPW)
    def _():
        pltpu.sync_copy(lab_hbm.at[pl.ds(start * CHUNK + CPW * CHUNK, CHUNK)],
                        laball.at[pl.ds(CPW * CHUNK, CHUNK)])

    def chunk_body(i, _):
        @pl.when(i < cnt)
        def _():
            row0 = (start + i) * CHUNK
            pltpu.sync_copy(x_hbm.at[pl.ds(row0, CHUNK)], xbuf)

            def group_body(g, _2):
                labv = laball[pl.ds(i * CHUNK + g * 16, 16)]
                offs = [labv[k] * G for k in range(16)]
                rbase = g * 16
                for j in range(GV):
                    xs = [xbuf[rbase + k, pl.ds(16 * j, 16)]
                          for k in range(16)]
                    for k in range(16):
                        plsc.addupdate(
                            accbuf.at[pl.ds(offs[k] + 16 * j, 16)], xs[k])
                    sq = _tree_sum([x * x for x in xs])
                    s = ssqbuf[pl.ds(16 * j, 16)]
                    ssqbuf[pl.ds(16 * j, 16)] = s + sq
                return 0

            lax.fori_loop(0, RG, group_body, 0)

        return 0

    lax.fori_loop(0, MAXC, chunk_body, 0)
    pltpu.sync_copy(accbuf, seg_out.at[wid])
    pltpu.sync_copy(ssqbuf, ssq_out.at[wid])


def _sc_apply_body(x_hbm, lab_hbm, tbl_hbm, out_hbm, xbuf, laball, tblbuf):
    wid = _worker_id()
    start, cnt = _my_chunks(wid)
    pltpu.sync_copy(tbl_hbm.at[0], tblbuf)
    pltpu.sync_copy(lab_hbm.at[pl.ds(start * CHUNK, CPW * CHUNK)], laball.at[pl.ds(0, CPW * CHUNK)])

    @pl.when(cnt > CPW)
    def _():
        pltpu.sync_copy(lab_hbm.at[pl.ds(start * CHUNK + CPW * CHUNK, CHUNK)],
                        laball.at[pl.ds(CPW * CHUNK, CHUNK)])

    def chunk_body(i, _):
        @pl.when(i < cnt)
        def _():
            row0 = (start + i) * CHUNK
            pltpu.sync_copy(x_hbm.at[pl.ds(row0, CHUNK)], xbuf)

            def group_body(g, _2):
                labv = laball[pl.ds(i * CHUNK + g * 16, 16)]
                offs = [labv[k] * G for k in range(16)]
                rbase = g * 16
                for j in range(GV):
                    for k in range(16):
                        v = tblbuf[pl.ds(offs[k] + 16 * j, 16)]
                        plsc.addupdate(
                            xbuf.at[rbase + k, pl.ds(16 * j, 16)], v)
                return 0

            lax.fori_loop(0, RG, group_body, 0)
            pltpu.sync_copy(xbuf, out_hbm.at[pl.ds(row0, CHUNK)])

        return 0

    lax.fori_loop(0, MAXC, chunk_body, 0)


def _finalize_body(segp_ref, ssqp_ref, lab_ref, tbl_ref):
    seg = jnp.sum(segp_ref[...], axis=0)
    ssq = jnp.sum(ssqp_ref[...], axis=0, keepdims=True)
    gm = jnp.sum(seg, axis=0, keepdims=True) / N
    gv = ssq / N - gm * gm
    inv_std = 1.0 / (jnp.sqrt(gv) + 1e-8)
    labs = lab_ref[:, 0, :]
    rows = []
    for b in range(NB):
        cnt = jnp.maximum(
            jnp.sum((labs == b).astype(jnp.float32)), 1.0)
        bm = seg[b:b + 1, :] / cnt
        rows.append(-(bm - gm) * inv_std)
    tbl_ref[...] = jnp.concatenate(rows, axis=1)


@jax.jit
def kernel(expression, batch_labels):
    mesh = plsc.VectorSubcoreMesh(core_axis_name="c", subcore_axis_name="s")

    stats = functools.partial(
        pl.kernel,
        mesh=mesh,
        out_type=[
            jax.ShapeDtypeStruct((NW, NB * G), jnp.float32),
            jax.ShapeDtypeStruct((NW, G), jnp.float32),
        ],
        scratch_types=[
            pltpu.VMEM((CHUNK, G), jnp.float32),
            pltpu.VMEM((MAXC * CHUNK,), jnp.int32),
            pltpu.VMEM((NB * G,), jnp.float32),
            pltpu.VMEM((G,), jnp.float32),
        ],
    )(_sc_stats_body)
    seg_p, ssq_p = stats(expression, batch_labels)
    seg_p = seg_p.reshape(NW, NB, G)

    labels3 = batch_labels.reshape(50, 1, N // 50)
    negtbl = pl.pallas_call(
        _finalize_body,
        out_shape=jax.ShapeDtypeStruct((1, NB * G), jnp.float32),
    )(seg_p, ssq_p, labels3)

    apply_fn = functools.partial(
        pl.kernel,
        mesh=mesh,
        out_type=jax.ShapeDtypeStruct((N, G), jnp.float32),
        scratch_types=[
            pltpu.VMEM((CHUNK, G), jnp.float32),
            pltpu.VMEM((MAXC * CHUNK,), jnp.int32),
            pltpu.VMEM((NB * G,), jnp.float32),
        ],
    )(_sc_apply_body)
    return apply_fn(expression, batch_labels, negtbl)


# speedup vs baseline: 3.2374x; 1.7328x over previous
import functools

import jax
import jax.numpy as jnp
from jax import lax
from jax.experimental import pallas as pl
from jax.experimental.pallas import tpu as pltpu
from jax.experimental.pallas import tpu_sc as plsc

NB = 8
N = 100000
G = 512
NW = 32
CHUNK = 160
NCHUNKS = N // CHUNK
CPW = NCHUNKS // NW
EXTRA = NCHUNKS % NW
MAXC = CPW + 1
GV = G // 16
RG = CHUNK // 16


def _worker_id():
    return lax.axis_index("s") * 2 + lax.axis_index("c")


def _my_chunks(wid):
    start = wid * CPW + jnp.minimum(wid, EXTRA)
    cnt = jnp.where(wid < EXTRA, CPW + 1, CPW)
    return start, cnt


def _tree_sum(vals):
    vals = list(vals)
    while len(vals) > 1:
        vals = [vals[i] + vals[i + 1] for i in range(0, len(vals) - 1, 2)] + (
            [vals[-1]] if len(vals) % 2 else [])
    return vals[0]


def _sc_stats_body(x_hbm, lab_hbm, seg_out, ssq_out, xbuf, laball, accbuf,
                   ssqbuf):
    wid = _worker_id()
    start, cnt = _my_chunks(wid)
    zero16 = jnp.zeros((16,), jnp.float32)
    for j in range(NB * GV):
        accbuf[pl.ds(16 * j, 16)] = zero16
    for j in range(GV):
        ssqbuf[pl.ds(16 * j, 16)] = zero16
    pltpu.sync_copy(lab_hbm.at[pl.ds(start * CHUNK, CPW * CHUNK)], laball.at[pl.ds(0, CPW * CHUNK)])

    @pl.when(cnt > CPW)
    def _():
        pltpu.sync_copy(lab_hbm.at[pl.ds(start * CHUNK + CPW * CHUNK, CHUNK)],
                        laball.at[pl.ds(CPW * CHUNK, CHUNK)])

    def chunk_body(i, _):
        @pl.when(i < cnt)
        def _():
            row0 = (start + i) * CHUNK
            pltpu.sync_copy(x_hbm.at[pl.ds(row0, CHUNK)], xbuf)

            def group_body(g, _2):
                labv = laball[pl.ds(i * CHUNK + g * 16, 16)]
                offs = [labv[k] * G for k in range(16)]
                rbase = g * 16
                for j in range(GV):
                    xs = [xbuf[rbase + k, pl.ds(16 * j, 16)]
                          for k in range(16)]
                    for k in range(16):
                        plsc.addupdate(
                            accbuf.at[pl.ds(offs[k] + 16 * j, 16)], xs[k])
                    sq = _tree_sum([x * x for x in xs])
                    s = ssqbuf[pl.ds(16 * j, 16)]
                    ssqbuf[pl.ds(16 * j, 16)] = s + sq
                return 0

            lax.fori_loop(0, RG, group_body, 0)

        return 0

    lax.fori_loop(0, MAXC, chunk_body, 0)
    pltpu.sync_copy(accbuf, seg_out.at[wid])
    pltpu.sync_copy(ssqbuf, ssq_out.at[wid])


def _sc_apply_body(x_hbm, lab_hbm, tbl_hbm, out_hbm, xbuf, laball, tblbuf):
    wid = _worker_id()
    start, cnt = _my_chunks(wid)
    pltpu.sync_copy(tbl_hbm.at[0], tblbuf)
    pltpu.sync_copy(lab_hbm.at[pl.ds(start * CHUNK, CPW * CHUNK)], laball.at[pl.ds(0, CPW * CHUNK)])

    @pl.when(cnt > CPW)
    def _():
        pltpu.sync_copy(lab_hbm.at[pl.ds(start * CHUNK + CPW * CHUNK, CHUNK)],
                        laball.at[pl.ds(CPW * CHUNK, CHUNK)])

    def chunk_body(i, _):
        @pl.when(i < cnt)
        def _():
            row0 = (start + i) * CHUNK
            pltpu.sync_copy(x_hbm.at[pl.ds(row0, CHUNK)], xbuf)

            def group_body(g, _2):
                labv = laball[pl.ds(i * CHUNK + g * 16, 16)]
                offs = [labv[k] * G for k in range(16)]
                rbase = g * 16
                for j in range(GV):
                    vs = [tblbuf[pl.ds(offs[k] + 16 * j, 16)]
                          for k in range(16)]
                    for k in range(16):
                        plsc.addupdate(
                            xbuf.at[rbase + k, pl.ds(16 * j, 16)], vs[k])
                return 0

            lax.fori_loop(0, RG, group_body, 0)
            pltpu.sync_copy(xbuf, out_hbm.at[pl.ds(row0, CHUNK)])

        return 0

    lax.fori_loop(0, MAXC, chunk_body, 0)


def _finalize_body(segp_ref, ssqp_ref, lab_ref, tbl_ref):
    seg = jnp.sum(segp_ref[...], axis=0)
    ssq = jnp.sum(ssqp_ref[...], axis=0, keepdims=True)
    gm = jnp.sum(seg, axis=0, keepdims=True) / N
    gv = ssq / N - gm * gm
    inv_std = 1.0 / (jnp.sqrt(gv) + 1e-8)
    labs = lab_ref[:, 0, :]
    rows = []
    for b in range(NB):
        cnt = jnp.maximum(
            jnp.sum((labs == b).astype(jnp.float32)), 1.0)
        bm = seg[b:b + 1, :] / cnt
        rows.append(-(bm - gm) * inv_std)
    tbl_ref[...] = jnp.concatenate(rows, axis=1)


@jax.jit
def kernel(expression, batch_labels):
    mesh = plsc.VectorSubcoreMesh(core_axis_name="c", subcore_axis_name="s")

    stats = functools.partial(
        pl.kernel,
        mesh=mesh,
        out_type=[
            jax.ShapeDtypeStruct((NW, NB * G), jnp.float32),
            jax.ShapeDtypeStruct((NW, G), jnp.float32),
        ],
        scratch_types=[
            pltpu.VMEM((CHUNK, G), jnp.float32),
            pltpu.VMEM((MAXC * CHUNK,), jnp.int32),
            pltpu.VMEM((NB * G,), jnp.float32),
            pltpu.VMEM((G,), jnp.float32),
        ],
    )(_sc_stats_body)
    seg_p, ssq_p = stats(expression, batch_labels)
    seg_p = seg_p.reshape(NW, NB, G)

    labels3 = batch_labels.reshape(50, 1, N // 50)
    negtbl = pl.pallas_call(
        _finalize_body,
        out_shape=jax.ShapeDtypeStruct((1, NB * G), jnp.float32),
    )(seg_p, ssq_p, labels3)

    apply_fn = functools.partial(
        pl.kernel,
        mesh=mesh,
        out_type=jax.ShapeDtypeStruct((N, G), jnp.float32),
        scratch_types=[
            pltpu.VMEM((CHUNK, G), jnp.float32),
            pltpu.VMEM((MAXC * CHUNK,), jnp.int32),
            pltpu.VMEM((NB * G,), jnp.float32),
        ],
    )(_sc_apply_body)
    return apply_fn(expression, batch_labels, negtbl)


# speedup vs baseline: 3.8807x; 1.1987x over previous
import functools

import jax
import jax.numpy as jnp
from jax import lax
from jax.experimental import pallas as pl
from jax.experimental.pallas import tpu as pltpu
from jax.experimental.pallas import tpu_sc as plsc

NB = 8
N = 100000
G = 512
NW = 32
CHUNK = 80
NCHUNKS = N // CHUNK
CPW = NCHUNKS // NW
EXTRA = NCHUNKS % NW
MAXC = CPW + 1
NPAIR = MAXC // 2
GV = G // 16
RG = CHUNK // 16


def _worker_id():
    return lax.axis_index("s") * 2 + lax.axis_index("c")


def _my_chunks(wid):
    start = wid * CPW + jnp.minimum(wid, EXTRA)
    cnt = jnp.where(wid < EXTRA, CPW + 1, CPW)
    return start, cnt


def _tree_sum(vals):
    vals = list(vals)
    while len(vals) > 1:
        vals = [vals[i] + vals[i + 1] for i in range(0, len(vals) - 1, 2)] + (
            [vals[-1]] if len(vals) % 2 else [])
    return vals[0]


def _load_labels(lab_hbm, laball, start, cnt):
    pltpu.sync_copy(lab_hbm.at[pl.ds(start * CHUNK, CPW * CHUNK)],
                    laball.at[pl.ds(0, CPW * CHUNK)])

    @pl.when(cnt > CPW)
    def _():
        pltpu.sync_copy(
            lab_hbm.at[pl.ds(start * CHUNK + CPW * CHUNK, CHUNK)],
            laball.at[pl.ds(CPW * CHUNK, CHUNK)])


def _sc_stats_body(x_hbm, lab_hbm, seg_out, ssq_out, xbuf0, xbuf1, laball,
                   accbuf, ssqbuf, sem0, sem1):
    wid = _worker_id()
    start, cnt = _my_chunks(wid)
    zero16 = jnp.zeros((16,), jnp.float32)
    for j in range(NB * GV):
        accbuf[pl.ds(16 * j, 16)] = zero16
    for j in range(GV):
        ssqbuf[pl.ds(16 * j, 16)] = zero16
    _load_labels(lab_hbm, laball, start, cnt)

    def _in(i, buf, sem):
        pltpu.async_copy(
            x_hbm.at[pl.ds((start + i) * CHUNK, CHUNK)], buf, sem)

    def _wait_in(i, buf, sem):
        pltpu.make_async_copy(
            x_hbm.at[pl.ds((start + i) * CHUNK, CHUNK)], buf, sem).wait()

    def _compute(i, buf):
        def group_body(g, _2):
            labv = laball[pl.ds(i * CHUNK + g * 16, 16)]
            offs = [labv[k] * G for k in range(16)]
            rbase = g * 16
            for j in range(GV):
                xs = [buf[rbase + k, pl.ds(16 * j, 16)] for k in range(16)]
                for k in range(16):
                    plsc.addupdate(
                        accbuf.at[pl.ds(offs[k] + 16 * j, 16)], xs[k])
                sq = _tree_sum([x * x for x in xs])
                s = ssqbuf[pl.ds(16 * j, 16)]
                ssqbuf[pl.ds(16 * j, 16)] = s + sq
            return 0

        lax.fori_loop(0, RG, group_body, 0)

    _in(0, xbuf0, sem0)
    _in(1, xbuf1, sem1)

    def pair_body(p, _):
        i0 = 2 * p
        i1 = 2 * p + 1
        _wait_in(i0, xbuf0, sem0)
        _compute(i0, xbuf0)

        @pl.when(i0 + 2 < cnt)
        def _():
            _in(i0 + 2, xbuf0, sem0)

        @pl.when(i1 < cnt)
        def _():
            _wait_in(i1, xbuf1, sem1)
            _compute(i1, xbuf1)

            @pl.when(i1 + 2 < cnt)
            def _():
                _in(i1 + 2, xbuf1, sem1)

        return 0

    lax.fori_loop(0, NPAIR, pair_body, 0)
    pltpu.sync_copy(accbuf, seg_out.at[wid])
    pltpu.sync_copy(ssqbuf, ssq_out.at[wid])


def _sc_apply_body(x_hbm, lab_hbm, tbl_hbm, out_hbm, xbuf0, xbuf1, laball,
                   tblbuf, si0, si1, so0, so1):
    wid = _worker_id()
    start, cnt = _my_chunks(wid)
    pltpu.sync_copy(tbl_hbm.at[0], tblbuf)
    _load_labels(lab_hbm, laball, start, cnt)

    def _in(i, buf, sem):
        pltpu.async_copy(
            x_hbm.at[pl.ds((start + i) * CHUNK, CHUNK)], buf, sem)

    def _wait_in(i, buf, sem):
        pltpu.make_async_copy(
            x_hbm.at[pl.ds((start + i) * CHUNK, CHUNK)], buf, sem).wait()

    def _out(i, buf, sem):
        pltpu.async_copy(
            buf, out_hbm.at[pl.ds((start + i) * CHUNK, CHUNK)], sem)

    def _wait_out(i, buf, sem):
        pltpu.make_async_copy(
            buf, out_hbm.at[pl.ds((start + i) * CHUNK, CHUNK)], sem).wait()

    def _compute(i, buf):
        def group_body(g, _2):
            labv = laball[pl.ds(i * CHUNK + g * 16, 16)]
            offs = [labv[k] * G for k in range(16)]
            rbase = g * 16
            for j in range(GV):
                vs = [tblbuf[pl.ds(offs[k] + 16 * j, 16)]
                      for k in range(16)]
                for k in range(16):
                    plsc.addupdate(
                        buf.at[rbase + k, pl.ds(16 * j, 16)], vs[k])
            return 0

        lax.fori_loop(0, RG, group_body, 0)

    _in(0, xbuf0, si0)
    _in(1, xbuf1, si1)

    def pair_body(p, _):
        i0 = 2 * p
        i1 = 2 * p + 1
        _wait_in(i0, xbuf0, si0)
        _compute(i0, xbuf0)
        _out(i0, xbuf0, so0)

        @pl.when(i1 < cnt)
        def _():
            _wait_in(i1, xbuf1, si1)
            _compute(i1, xbuf1)
            _out(i1, xbuf1, so1)

        @pl.when(i0 + 2 < cnt)
        def _():
            _wait_out(i0, xbuf0, so0)
            _in(i0 + 2, xbuf0, si0)

        @pl.when(i1 + 2 < cnt)
        def _():
            _wait_out(i1, xbuf1, so1)
            _in(i1 + 2, xbuf1, si1)

        return 0

    lax.fori_loop(0, NPAIR, pair_body, 0)
    _wait_out(0, xbuf0, so0)
    _wait_out(0, xbuf1, so1)


def _finalize_body(segp_ref, ssqp_ref, lab_ref, tbl_ref):
    seg = jnp.sum(segp_ref[...], axis=0)
    ssq = jnp.sum(ssqp_ref[...], axis=0, keepdims=True)
    gm = jnp.sum(seg, axis=0, keepdims=True) / N
    gv = ssq / N - gm * gm
    inv_std = 1.0 / (jnp.sqrt(gv) + 1e-8)
    labs = lab_ref[:, 0, :]
    rows = []
    for b in range(NB):
        cnt = jnp.maximum(
            jnp.sum((labs == b).astype(jnp.float32)), 1.0)
        bm = seg[b:b + 1, :] / cnt
        rows.append(-(bm - gm) * inv_std)
    tbl_ref[...] = jnp.concatenate(rows, axis=1)


@jax.jit
def kernel(expression, batch_labels):
    mesh = plsc.VectorSubcoreMesh(core_axis_name="c", subcore_axis_name="s")

    stats = functools.partial(
        pl.kernel,
        mesh=mesh,
        out_type=[
            jax.ShapeDtypeStruct((NW, NB * G), jnp.float32),
            jax.ShapeDtypeStruct((NW, G), jnp.float32),
        ],
        scratch_types=[
            pltpu.VMEM((CHUNK, G), jnp.float32),
            pltpu.VMEM((CHUNK, G), jnp.float32),
            pltpu.VMEM((MAXC * CHUNK,), jnp.int32),
            pltpu.VMEM((NB * G,), jnp.float32),
            pltpu.VMEM((G,), jnp.float32),
            pltpu.SemaphoreType.DMA,
            pltpu.SemaphoreType.DMA,
        ],
    )(_sc_stats_body)
    seg_p, ssq_p = stats(expression, batch_labels)
    seg_p = seg_p.reshape(NW, NB, G)

    labels3 = batch_labels.reshape(50, 1, N // 50)
    negtbl = pl.pallas_call(
        _finalize_body,
        out_shape=jax.ShapeDtypeStruct((1, NB * G), jnp.float32),
    )(seg_p, ssq_p, labels3)

    apply_fn = functools.partial(
        pl.kernel,
        mesh=mesh,
        out_type=jax.ShapeDtypeStruct((N, G), jnp.float32),
        scratch_types=[
            pltpu.VMEM((CHUNK, G), jnp.float32),
            pltpu.VMEM((CHUNK, G), jnp.float32),
            pltpu.VMEM((MAXC * CHUNK,), jnp.int32),
            pltpu.VMEM((NB * G,), jnp.float32),
            pltpu.SemaphoreType.DMA,
            pltpu.SemaphoreType.DMA,
            pltpu.SemaphoreType.DMA,
            pltpu.SemaphoreType.DMA,
        ],
    )(_sc_apply_body)
    return apply_fn(expression, batch_labels, negtbl)


# speedup vs baseline: 5.7231x; 1.4748x over previous
import functools

import jax
import jax.numpy as jnp
from jax import lax
from jax.experimental import pallas as pl
from jax.experimental.pallas import tpu as pltpu
from jax.experimental.pallas import tpu_sc as plsc

NB = 8
N = 100000
G = 512
NW = 32
CHUNK = 80
NCHUNKS = N // CHUNK
CPW = NCHUNKS // NW
EXTRA = NCHUNKS % NW
MAXC = CPW + 1
NPAIR = MAXC // 2
TC_BLK = 2000
TC_ROWS = 70000
SC_CHUNK0 = TC_ROWS // CHUNK
NCHUNKS_S = (N - TC_ROWS) // CHUNK
CPW_S = NCHUNKS_S // NW
EXTRA_S = NCHUNKS_S % NW
MAXC_S = CPW_S + 1
NPAIR_S = MAXC_S // 2
GV = G // 16
RG = CHUNK // 16


def _worker_id():
    return lax.axis_index("s") * 2 + lax.axis_index("c")


def _my_chunks(wid, chunk0=0, cpw=CPW, extra=EXTRA):
    start = chunk0 + wid * cpw + jnp.minimum(wid, extra)
    cnt = jnp.where(wid < extra, cpw + 1, cpw)
    return start, cnt


def _tree_sum(vals):
    vals = list(vals)
    while len(vals) > 1:
        vals = [vals[i] + vals[i + 1] for i in range(0, len(vals) - 1, 2)] + (
            [vals[-1]] if len(vals) % 2 else [])
    return vals[0]


def _load_labels(lab_hbm, laball, start, cnt, cpw=CPW):
    pltpu.sync_copy(lab_hbm.at[pl.ds(start * CHUNK, cpw * CHUNK)],
                    laball.at[pl.ds(0, cpw * CHUNK)])

    @pl.when(cnt > cpw)
    def _():
        pltpu.sync_copy(
            lab_hbm.at[pl.ds(start * CHUNK + cpw * CHUNK, CHUNK)],
            laball.at[pl.ds(cpw * CHUNK, CHUNK)])


def _sc_stats_body(x_hbm, lab_hbm, seg_out, ssq_out, xbuf0, xbuf1, laball,
                   accbuf, ssqbuf, sem0, sem1):
    wid = _worker_id()
    start, cnt = _my_chunks(wid, SC_CHUNK0, CPW_S, EXTRA_S)
    zero16 = jnp.zeros((16,), jnp.float32)
    for j in range(NB * GV):
        accbuf[pl.ds(16 * j, 16)] = zero16
    for j in range(GV):
        ssqbuf[pl.ds(16 * j, 16)] = zero16
    _load_labels(lab_hbm, laball, start, cnt, CPW_S)

    def _in(i, buf, sem):
        pltpu.async_copy(
            x_hbm.at[pl.ds((start + i) * CHUNK, CHUNK)], buf, sem)

    def _wait_in(i, buf, sem):
        pltpu.make_async_copy(
            x_hbm.at[pl.ds((start + i) * CHUNK, CHUNK)], buf, sem).wait()

    def _compute(i, buf):
        def group_body(g, _2):
            labv = laball[pl.ds(i * CHUNK + g * 16, 16)]
            offs = [labv[k] * G for k in range(16)]
            rbase = g * 16
            for j in range(GV):
                xs = [buf[rbase + k, pl.ds(16 * j, 16)] for k in range(16)]
                for k in range(16):
                    plsc.addupdate(
                        accbuf.at[pl.ds(offs[k] + 16 * j, 16)], xs[k])
                sq = _tree_sum([x * x for x in xs])
                s = ssqbuf[pl.ds(16 * j, 16)]
                ssqbuf[pl.ds(16 * j, 16)] = s + sq
            return 0

        lax.fori_loop(0, RG, group_body, 0)

    _in(0, xbuf0, sem0)
    _in(1, xbuf1, sem1)

    def pair_body(p, _):
        i0 = 2 * p
        i1 = 2 * p + 1
        _wait_in(i0, xbuf0, sem0)
        _compute(i0, xbuf0)

        @pl.when(i0 + 2 < cnt)
        def _():
            _in(i0 + 2, xbuf0, sem0)

        @pl.when(i1 < cnt)
        def _():
            _wait_in(i1, xbuf1, sem1)
            _compute(i1, xbuf1)

            @pl.when(i1 + 2 < cnt)
            def _():
                _in(i1 + 2, xbuf1, sem1)

        return 0

    lax.fori_loop(0, NPAIR_S, pair_body, 0)
    pltpu.sync_copy(accbuf, seg_out.at[wid])
    pltpu.sync_copy(ssqbuf, ssq_out.at[wid])


def _sc_apply_body(x_hbm, lab_hbm, tbl_hbm, out_hbm, xbuf0, xbuf1, laball,
                   tblbuf, si0, si1, so0, so1):
    wid = _worker_id()
    start, cnt = _my_chunks(wid)
    pltpu.sync_copy(tbl_hbm.at[0], tblbuf)
    _load_labels(lab_hbm, laball, start, cnt)

    def _in(i, buf, sem):
        pltpu.async_copy(
            x_hbm.at[pl.ds((start + i) * CHUNK, CHUNK)], buf, sem)

    def _wait_in(i, buf, sem):
        pltpu.make_async_copy(
            x_hbm.at[pl.ds((start + i) * CHUNK, CHUNK)], buf, sem).wait()

    def _out(i, buf, sem):
        pltpu.async_copy(
            buf, out_hbm.at[pl.ds((start + i) * CHUNK, CHUNK)], sem)

    def _wait_out(i, buf, sem):
        pltpu.make_async_copy(
            buf, out_hbm.at[pl.ds((start + i) * CHUNK, CHUNK)], sem).wait()

    def _compute(i, buf):
        def group_body(g, _2):
            labv = laball[pl.ds(i * CHUNK + g * 16, 16)]
            offs = [labv[k] * G for k in range(16)]
            rbase = g * 16
            for j in range(GV):
                vs = [tblbuf[pl.ds(offs[k] + 16 * j, 16)]
                      for k in range(16)]
                for k in range(16):
                    plsc.addupdate(
                        buf.at[rbase + k, pl.ds(16 * j, 16)], vs[k])
            return 0

        lax.fori_loop(0, RG, group_body, 0)

    _in(0, xbuf0, si0)
    _in(1, xbuf1, si1)

    def pair_body(p, _):
        i0 = 2 * p
        i1 = 2 * p + 1
        _wait_in(i0, xbuf0, si0)
        _compute(i0, xbuf0)
        _out(i0, xbuf0, so0)

        @pl.when(i1 < cnt)
        def _():
            _wait_in(i1, xbuf1, si1)
            _compute(i1, xbuf1)
            _out(i1, xbuf1, so1)

        @pl.when(i0 + 2 < cnt)
        def _():
            _wait_out(i0, xbuf0, so0)
            _in(i0 + 2, xbuf0, si0)

        @pl.when(i1 + 2 < cnt)
        def _():
            _wait_out(i1, xbuf1, so1)
            _in(i1 + 2, xbuf1, si1)

        return 0

    lax.fori_loop(0, NPAIR, pair_body, 0)
    _wait_out(0, xbuf0, so0)
    _wait_out(0, xbuf1, so1)


def _tc_stats_body(x_ref, lab_ref, seg_ref, ssq_ref):
    @pl.when(pl.program_id(0) == 0)
    def _():
        seg_ref[...] = jnp.zeros_like(seg_ref)
        ssq_ref[...] = jnp.zeros_like(ssq_ref)

    x = x_ref[...]
    labels = lab_ref[0, 0, :]
    onehot = (labels[:, None] == lax.broadcasted_iota(jnp.int32, (1, NB), 1)
              ).astype(jnp.float32)
    seg_ref[...] += lax.dot_general(
        onehot, x, (((0,), (0,)), ((), ())),
        preferred_element_type=jnp.float32)
    ssq_ref[...] += jnp.sum(x * x, axis=0, keepdims=True)


def _finalize_body(segp_ref, ssqp_ref, segtc_ref, ssqtc_ref, lab_ref,
                   tbl_ref):
    seg = jnp.sum(segp_ref[...], axis=0) + segtc_ref[...]
    ssq = jnp.sum(ssqp_ref[...], axis=0, keepdims=True) + ssqtc_ref[...]
    gm = jnp.sum(seg, axis=0, keepdims=True) / N
    gv = ssq / N - gm * gm
    inv_std = 1.0 / (jnp.sqrt(gv) + 1e-8)
    labs = lab_ref[:, 0, :]
    rows = []
    for b in range(NB):
        cnt = jnp.maximum(
            jnp.sum((labs == b).astype(jnp.float32)), 1.0)
        bm = seg[b:b + 1, :] / cnt
        rows.append(-(bm - gm) * inv_std)
    tbl_ref[...] = jnp.concatenate(rows, axis=1)


@jax.jit
def kernel(expression, batch_labels):
    mesh = plsc.VectorSubcoreMesh(core_axis_name="c", subcore_axis_name="s")

    stats = functools.partial(
        pl.kernel,
        mesh=mesh,
        out_type=[
            jax.ShapeDtypeStruct((NW, NB * G), jnp.float32),
            jax.ShapeDtypeStruct((NW, G), jnp.float32),
        ],
        scratch_types=[
            pltpu.VMEM((CHUNK, G), jnp.float32),
            pltpu.VMEM((CHUNK, G), jnp.float32),
            pltpu.VMEM((MAXC * CHUNK,), jnp.int32),
            pltpu.VMEM((NB * G,), jnp.float32),
            pltpu.VMEM((G,), jnp.float32),
            pltpu.SemaphoreType.DMA,
            pltpu.SemaphoreType.DMA,
        ],
    )(_sc_stats_body)
    seg_p, ssq_p = stats(expression, batch_labels)
    seg_p = seg_p.reshape(NW, NB, G)

    labels3 = batch_labels.reshape(N // TC_BLK, 1, TC_BLK)
    seg_tc, ssq_tc = pl.pallas_call(
        _tc_stats_body,
        grid=(TC_ROWS // TC_BLK,),
        in_specs=[
            pl.BlockSpec((TC_BLK, G), lambda i: (i, 0)),
            pl.BlockSpec((1, 1, TC_BLK), lambda i: (i, 0, 0)),
        ],
        out_specs=[
            pl.BlockSpec((NB, G), lambda i: (0, 0)),
            pl.BlockSpec((1, G), lambda i: (0, 0)),
        ],
        out_shape=[
            jax.ShapeDtypeStruct((NB, G), jnp.float32),
            jax.ShapeDtypeStruct((1, G), jnp.float32),
        ],
    )(expression, labels3)

    negtbl = pl.pallas_call(
        _finalize_body,
        out_shape=jax.ShapeDtypeStruct((1, NB * G), jnp.float32),
    )(seg_p, ssq_p, seg_tc, ssq_tc, labels3)

    apply_fn = functools.partial(
        pl.kernel,
        mesh=mesh,
        out_type=jax.ShapeDtypeStruct((N, G), jnp.float32),
        scratch_types=[
            pltpu.VMEM((CHUNK, G), jnp.float32),
            pltpu.VMEM((CHUNK, G), jnp.float32),
            pltpu.VMEM((MAXC * CHUNK,), jnp.int32),
            pltpu.VMEM((NB * G,), jnp.float32),
            pltpu.SemaphoreType.DMA,
            pltpu.SemaphoreType.DMA,
            pltpu.SemaphoreType.DMA,
            pltpu.SemaphoreType.DMA,
        ],
    )(_sc_apply_body)
    return apply_fn(expression, batch_labels, negtbl)


# speedup vs baseline: 6.2283x; 1.0883x over previous
import functools

import jax
import jax.numpy as jnp
from jax import lax
from jax.experimental import pallas as pl
from jax.experimental.pallas import tpu as pltpu
from jax.experimental.pallas import tpu_sc as plsc

NB = 8
N = 100000
G = 512
NW = 32
CHUNK = 80
NCHUNKS = N // CHUNK
CPW = NCHUNKS // NW
EXTRA = NCHUNKS % NW
MAXC = CPW + 1
NPAIR = MAXC // 2
TC_BLK = 2000
TC_ROWS = 70000
SC_CHUNK0 = TC_ROWS // CHUNK
NCHUNKS_S = (N - TC_ROWS) // CHUNK
CPW_S = NCHUNKS_S // NW
EXTRA_S = NCHUNKS_S % NW
MAXC_S = CPW_S + 1
NPAIR_S = MAXC_S // 2
TC_AROWS = 50000
SC_ACHUNK0 = TC_AROWS // CHUNK
NCHUNKS_A = (N - TC_AROWS) // CHUNK
CPW_A = NCHUNKS_A // NW
EXTRA_A = NCHUNKS_A % NW
MAXC_A = CPW_A + 1
NPAIR_A = MAXC_A // 2
GV = G // 16
RG = CHUNK // 16


def _worker_id():
    return lax.axis_index("s") * 2 + lax.axis_index("c")


def _my_chunks(wid, chunk0=0, cpw=CPW, extra=EXTRA):
    start = chunk0 + wid * cpw + jnp.minimum(wid, extra)
    cnt = jnp.where(wid < extra, cpw + 1, cpw)
    return start, cnt


def _tree_sum(vals):
    vals = list(vals)
    while len(vals) > 1:
        vals = [vals[i] + vals[i + 1] for i in range(0, len(vals) - 1, 2)] + (
            [vals[-1]] if len(vals) % 2 else [])
    return vals[0]


def _load_labels(lab_hbm, laball, start, cnt, cpw=CPW):
    pltpu.sync_copy(lab_hbm.at[pl.ds(start * CHUNK, cpw * CHUNK)],
                    laball.at[pl.ds(0, cpw * CHUNK)])

    @pl.when(cnt > cpw)
    def _():
        pltpu.sync_copy(
            lab_hbm.at[pl.ds(start * CHUNK + cpw * CHUNK, CHUNK)],
            laball.at[pl.ds(cpw * CHUNK, CHUNK)])


def _sc_stats_body(x_hbm, lab_hbm, seg_out, ssq_out, xbuf0, xbuf1, laball,
                   accbuf, ssqbuf, sem0, sem1):
    wid = _worker_id()
    start, cnt = _my_chunks(wid, SC_CHUNK0, CPW_S, EXTRA_S)
    zero16 = jnp.zeros((16,), jnp.float32)
    for j in range(NB * GV):
        accbuf[pl.ds(16 * j, 16)] = zero16
    for j in range(GV):
        ssqbuf[pl.ds(16 * j, 16)] = zero16
    _load_labels(lab_hbm, laball, start, cnt, CPW_S)

    def _in(i, buf, sem):
        pltpu.async_copy(
            x_hbm.at[pl.ds((start + i) * CHUNK, CHUNK)], buf, sem)

    def _wait_in(i, buf, sem):
        pltpu.make_async_copy(
            x_hbm.at[pl.ds((start + i) * CHUNK, CHUNK)], buf, sem).wait()

    def _compute(i, buf):
        def group_body(g, _2):
            labv = laball[pl.ds(i * CHUNK + g * 16, 16)]
            offs = [labv[k] * G for k in range(16)]
            rbase = g * 16
            for j in range(GV):
                xs = [buf[rbase + k, pl.ds(16 * j, 16)] for k in range(16)]
                for k in range(16):
                    plsc.addupdate(
                        accbuf.at[pl.ds(offs[k] + 16 * j, 16)], xs[k])
                sq = _tree_sum([x * x for x in xs])
                s = ssqbuf[pl.ds(16 * j, 16)]
                ssqbuf[pl.ds(16 * j, 16)] = s + sq
            return 0

        lax.fori_loop(0, RG, group_body, 0)

    _in(0, xbuf0, sem0)
    _in(1, xbuf1, sem1)

    def pair_body(p, _):
        i0 = 2 * p
        i1 = 2 * p + 1
        _wait_in(i0, xbuf0, sem0)
        _compute(i0, xbuf0)

        @pl.when(i0 + 2 < cnt)
        def _():
            _in(i0 + 2, xbuf0, sem0)

        @pl.when(i1 < cnt)
        def _():
            _wait_in(i1, xbuf1, sem1)
            _compute(i1, xbuf1)

            @pl.when(i1 + 2 < cnt)
            def _():
                _in(i1 + 2, xbuf1, sem1)

        return 0

    lax.fori_loop(0, NPAIR_S, pair_body, 0)
    pltpu.sync_copy(accbuf, seg_out.at[wid])
    pltpu.sync_copy(ssqbuf, ssq_out.at[wid])


def _sc_apply_body(x_hbm, lab_hbm, tbl_hbm, out_hbm, xbuf0, xbuf1, laball,
                   tblbuf, si0, si1, so0, so1):
    wid = _worker_id()
    start, cnt = _my_chunks(wid, SC_ACHUNK0, CPW_A, EXTRA_A)
    pltpu.sync_copy(tbl_hbm.at[0], tblbuf)
    _load_labels(lab_hbm, laball, start, cnt, CPW_A)

    def _in(i, buf, sem):
        pltpu.async_copy(
            x_hbm.at[pl.ds((start + i) * CHUNK, CHUNK)], buf, sem)

    def _wait_in(i, buf, sem):
        pltpu.make_async_copy(
            x_hbm.at[pl.ds((start + i) * CHUNK, CHUNK)], buf, sem).wait()

    def _out(i, buf, sem):
        pltpu.async_copy(
            buf, out_hbm.at[pl.ds((start + i) * CHUNK, CHUNK)], sem)

    def _wait_out(i, buf, sem):
        pltpu.make_async_copy(
            buf, out_hbm.at[pl.ds((start + i) * CHUNK, CHUNK)], sem).wait()

    def _compute(i, buf):
        def group_body(g, _2):
            labv = laball[pl.ds(i * CHUNK + g * 16, 16)]
            offs = [labv[k] * G for k in range(16)]
            rbase = g * 16
            for j in range(GV):
                vs = [tblbuf[pl.ds(offs[k] + 16 * j, 16)]
                      for k in range(16)]
                for k in range(16):
                    plsc.addupdate(
                        buf.at[rbase + k, pl.ds(16 * j, 16)], vs[k])
            return 0

        lax.fori_loop(0, RG, group_body, 0)

    _in(0, xbuf0, si0)
    _in(1, xbuf1, si1)

    def pair_body(p, _):
        i0 = 2 * p
        i1 = 2 * p + 1
        _wait_in(i0, xbuf0, si0)
        _compute(i0, xbuf0)
        _out(i0, xbuf0, so0)

        @pl.when(i1 < cnt)
        def _():
            _wait_in(i1, xbuf1, si1)
            _compute(i1, xbuf1)
            _out(i1, xbuf1, so1)

        @pl.when(i0 + 2 < cnt)
        def _():
            _wait_out(i0, xbuf0, so0)
            _in(i0 + 2, xbuf0, si0)

        @pl.when(i1 + 2 < cnt)
        def _():
            _wait_out(i1, xbuf1, so1)
            _in(i1 + 2, xbuf1, si1)

        return 0

    lax.fori_loop(0, NPAIR_A, pair_body, 0)
    _wait_out(0, xbuf0, so0)
    _wait_out(0, xbuf1, so1)


def _tc_stats_body(x_ref, lab_ref, seg_ref, ssq_ref):
    @pl.when(pl.program_id(0) == 0)
    def _():
        seg_ref[...] = jnp.zeros_like(seg_ref)
        ssq_ref[...] = jnp.zeros_like(ssq_ref)

    x = x_ref[...]
    labels = lab_ref[0, 0, :]
    onehot = (labels[:, None] == lax.broadcasted_iota(jnp.int32, (1, NB), 1)
              ).astype(jnp.float32)
    seg_ref[...] += lax.dot_general(
        onehot, x, (((0,), (0,)), ((), ())),
        preferred_element_type=jnp.float32)
    ssq_ref[...] += jnp.sum(x * x, axis=0, keepdims=True)


def _tc_apply_body(alias_ref, x_ref, lab_ref, tbl_ref, out_ref):
    del alias_ref
    labels = lab_ref[0, 0, :]
    onehot = (labels[:, None] == lax.broadcasted_iota(jnp.int32, (1, NB), 1)
              ).astype(jnp.float32)
    corr = lax.dot_general(
        onehot, tbl_ref[...], (((1,), (0,)), ((), ())),
        preferred_element_type=jnp.float32)
    out_ref[...] = x_ref[...] + corr


def _finalize_body(segp_ref, ssqp_ref, segtc_ref, ssqtc_ref, lab_ref,
                   tbl_ref):
    seg = jnp.sum(segp_ref[...], axis=0) + segtc_ref[...]
    ssq = jnp.sum(ssqp_ref[...], axis=0, keepdims=True) + ssqtc_ref[...]
    gm = jnp.sum(seg, axis=0, keepdims=True) / N
    gv = ssq / N - gm * gm
    inv_std = 1.0 / (jnp.sqrt(gv) + 1e-8)
    labs = lab_ref[:, 0, :]
    rows = []
    for b in range(NB):
        cnt = jnp.maximum(
            jnp.sum((labs == b).astype(jnp.float32)), 1.0)
        bm = seg[b:b + 1, :] / cnt
        rows.append(-(bm - gm) * inv_std)
    tbl_ref[...] = jnp.concatenate(rows, axis=1)


@jax.jit
def kernel(expression, batch_labels):
    mesh = plsc.VectorSubcoreMesh(core_axis_name="c", subcore_axis_name="s")

    stats = functools.partial(
        pl.kernel,
        mesh=mesh,
        out_type=[
            jax.ShapeDtypeStruct((NW, NB * G), jnp.float32),
            jax.ShapeDtypeStruct((NW, G), jnp.float32),
        ],
        scratch_types=[
            pltpu.VMEM((CHUNK, G), jnp.float32),
            pltpu.VMEM((CHUNK, G), jnp.float32),
            pltpu.VMEM((MAXC * CHUNK,), jnp.int32),
            pltpu.VMEM((NB * G,), jnp.float32),
            pltpu.VMEM((G,), jnp.float32),
            pltpu.SemaphoreType.DMA,
            pltpu.SemaphoreType.DMA,
        ],
    )(_sc_stats_body)
    seg_p, ssq_p = stats(expression, batch_labels)
    seg_p = seg_p.reshape(NW, NB, G)

    labels3 = batch_labels.reshape(N // TC_BLK, 1, TC_BLK)
    seg_tc, ssq_tc = pl.pallas_call(
        _tc_stats_body,
        grid=(TC_ROWS // TC_BLK,),
        in_specs=[
            pl.BlockSpec((TC_BLK, G), lambda i: (i, 0)),
            pl.BlockSpec((1, 1, TC_BLK), lambda i: (i, 0, 0)),
        ],
        out_specs=[
            pl.BlockSpec((NB, G), lambda i: (0, 0)),
            pl.BlockSpec((1, G), lambda i: (0, 0)),
        ],
        out_shape=[
            jax.ShapeDtypeStruct((NB, G), jnp.float32),
            jax.ShapeDtypeStruct((1, G), jnp.float32),
        ],
    )(expression, labels3)

    negtbl = pl.pallas_call(
        _finalize_body,
        out_shape=jax.ShapeDtypeStruct((1, NB * G), jnp.float32),
    )(seg_p, ssq_p, seg_tc, ssq_tc, labels3)

    apply_fn = functools.partial(
        pl.kernel,
        mesh=mesh,
        out_type=jax.ShapeDtypeStruct((N, G), jnp.float32),
        scratch_types=[
            pltpu.VMEM((CHUNK, G), jnp.float32),
            pltpu.VMEM((CHUNK, G), jnp.float32),
            pltpu.VMEM((MAXC_A * CHUNK,), jnp.int32),
            pltpu.VMEM((NB * G,), jnp.float32),
            pltpu.SemaphoreType.DMA,
            pltpu.SemaphoreType.DMA,
            pltpu.SemaphoreType.DMA,
            pltpu.SemaphoreType.DMA,
        ],
    )(_sc_apply_body)
    out_sc = apply_fn(expression, batch_labels, negtbl)

    negtbl2d = negtbl.reshape(NB, G)
    return pl.pallas_call(
        _tc_apply_body,
        grid=(TC_AROWS // TC_BLK,),
        in_specs=[
            pl.BlockSpec(memory_space=pl.ANY),
            pl.BlockSpec((TC_BLK, G), lambda i: (i, 0)),
            pl.BlockSpec((1, 1, TC_BLK), lambda i: (i, 0, 0)),
            pl.BlockSpec((NB, G), lambda i: (0, 0)),
        ],
        out_specs=pl.BlockSpec((TC_BLK, G), lambda i: (i, 0)),
        out_shape=jax.ShapeDtypeStruct((N, G), jnp.float32),
        input_output_aliases={0: 0},
    )(out_sc, expression, labels3, negtbl2d)


# speedup vs baseline: 6.6460x; 1.0671x over previous
import functools

import jax
import jax.numpy as jnp
from jax import lax
from jax.experimental import pallas as pl
from jax.experimental.pallas import tpu as pltpu
from jax.experimental.pallas import tpu_sc as plsc

NB = 8
N = 100000
G = 512
NW = 32
CHUNK = 80
NCHUNKS = N // CHUNK
CPW = NCHUNKS // NW
EXTRA = NCHUNKS % NW
MAXC = CPW + 1
NPAIR = MAXC // 2
TC_BLK = 2000
TC_ROWS = 76000
SC_CHUNK0 = TC_ROWS // CHUNK
NCHUNKS_S = (N - TC_ROWS) // CHUNK
CPW_S = NCHUNKS_S // NW
EXTRA_S = NCHUNKS_S % NW
MAXC_S = CPW_S + 1
NPAIR_S = MAXC_S // 2
TC_AROWS = 60000
SC_ACHUNK0 = TC_AROWS // CHUNK
NCHUNKS_A = (N - TC_AROWS) // CHUNK
CPW_A = NCHUNKS_A // NW
EXTRA_A = NCHUNKS_A % NW
MAXC_A = CPW_A + 1
NPAIR_A = MAXC_A // 2
GV = G // 16
RG = CHUNK // 16


def _worker_id():
    return lax.axis_index("s") * 2 + lax.axis_index("c")


def _my_chunks(wid, chunk0=0, cpw=CPW, extra=EXTRA):
    start = chunk0 + wid * cpw + jnp.minimum(wid, extra)
    cnt = jnp.where(wid < extra, cpw + 1, cpw)
    return start, cnt


def _tree_sum(vals):
    vals = list(vals)
    while len(vals) > 1:
        vals = [vals[i] + vals[i + 1] for i in range(0, len(vals) - 1, 2)] + (
            [vals[-1]] if len(vals) % 2 else [])
    return vals[0]


def _load_labels(lab_hbm, laball, start, cnt, cpw=CPW):
    pltpu.sync_copy(lab_hbm.at[pl.ds(start * CHUNK, cpw * CHUNK)],
                    laball.at[pl.ds(0, cpw * CHUNK)])

    @pl.when(cnt > cpw)
    def _():
        pltpu.sync_copy(
            lab_hbm.at[pl.ds(start * CHUNK + cpw * CHUNK, CHUNK)],
            laball.at[pl.ds(cpw * CHUNK, CHUNK)])


def _sc_stats_body(x_hbm, lab_hbm, seg_out, ssq_out, xbuf0, xbuf1, laball,
                   accbuf, ssqbuf, sem0, sem1):
    wid = _worker_id()
    start, cnt = _my_chunks(wid, SC_CHUNK0, CPW_S, EXTRA_S)
    zero16 = jnp.zeros((16,), jnp.float32)
    for j in range(NB * GV):
        accbuf[pl.ds(16 * j, 16)] = zero16
    for j in range(GV):
        ssqbuf[pl.ds(16 * j, 16)] = zero16
    _load_labels(lab_hbm, laball, start, cnt, CPW_S)

    def _in(i, buf, sem):
        pltpu.async_copy(
            x_hbm.at[pl.ds((start + i) * CHUNK, CHUNK)], buf, sem)

    def _wait_in(i, buf, sem):
        pltpu.make_async_copy(
            x_hbm.at[pl.ds((start + i) * CHUNK, CHUNK)], buf, sem).wait()

    def _compute(i, buf):
        def group_body(g, _2):
            labv = laball[pl.ds(i * CHUNK + g * 16, 16)]
            offs = [labv[k] * G for k in range(16)]
            rbase = g * 16
            for j in range(GV):
                xs = [buf[rbase + k, pl.ds(16 * j, 16)] for k in range(16)]
                for k in range(16):
                    plsc.addupdate(
                        accbuf.at[pl.ds(offs[k] + 16 * j, 16)], xs[k])
                sq = _tree_sum([x * x for x in xs])
                s = ssqbuf[pl.ds(16 * j, 16)]
                ssqbuf[pl.ds(16 * j, 16)] = s + sq
            return 0

        lax.fori_loop(0, RG, group_body, 0)

    _in(0, xbuf0, sem0)
    _in(1, xbuf1, sem1)

    def pair_body(p, _):
        i0 = 2 * p
        i1 = 2 * p + 1
        _wait_in(i0, xbuf0, sem0)
        _compute(i0, xbuf0)

        @pl.when(i0 + 2 < cnt)
        def _():
            _in(i0 + 2, xbuf0, sem0)

        @pl.when(i1 < cnt)
        def _():
            _wait_in(i1, xbuf1, sem1)
            _compute(i1, xbuf1)

            @pl.when(i1 + 2 < cnt)
            def _():
                _in(i1 + 2, xbuf1, sem1)

        return 0

    lax.fori_loop(0, NPAIR_S, pair_body, 0)
    pltpu.sync_copy(accbuf, seg_out.at[wid])
    pltpu.sync_copy(ssqbuf, ssq_out.at[wid])


def _sc_apply_body(x_hbm, lab_hbm, tbl_hbm, out_hbm, xbuf0, xbuf1, laball,
                   tblbuf, si0, si1, so0, so1):
    wid = _worker_id()
    start, cnt = _my_chunks(wid, SC_ACHUNK0, CPW_A, EXTRA_A)
    pltpu.sync_copy(tbl_hbm.at[0], tblbuf)
    _load_labels(lab_hbm, laball, start, cnt, CPW_A)

    def _in(i, buf, sem):
        pltpu.async_copy(
            x_hbm.at[pl.ds((start + i) * CHUNK, CHUNK)], buf, sem)

    def _wait_in(i, buf, sem):
        pltpu.make_async_copy(
            x_hbm.at[pl.ds((start + i) * CHUNK, CHUNK)], buf, sem).wait()

    def _out(i, buf, sem):
        pltpu.async_copy(
            buf, out_hbm.at[pl.ds((start + i) * CHUNK, CHUNK)], sem)

    def _wait_out(i, buf, sem):
        pltpu.make_async_copy(
            buf, out_hbm.at[pl.ds((start + i) * CHUNK, CHUNK)], sem).wait()

    def _compute(i, buf):
        def group_body(g, _2):
            labv = laball[pl.ds(i * CHUNK + g * 16, 16)]
            offs = [labv[k] * G for k in range(16)]
            rbase = g * 16
            for j in range(GV):
                vs = [tblbuf[pl.ds(offs[k] + 16 * j, 16)]
                      for k in range(16)]
                for k in range(16):
                    plsc.addupdate(
                        buf.at[rbase + k, pl.ds(16 * j, 16)], vs[k])
            return 0

        lax.fori_loop(0, RG, group_body, 0)

    _in(0, xbuf0, si0)
    _in(1, xbuf1, si1)

    def pair_body(p, _):
        i0 = 2 * p
        i1 = 2 * p + 1
        _wait_in(i0, xbuf0, si0)
        _compute(i0, xbuf0)
        _out(i0, xbuf0, so0)

        @pl.when(i1 < cnt)
        def _():
            _wait_in(i1, xbuf1, si1)
            _compute(i1, xbuf1)
            _out(i1, xbuf1, so1)

        @pl.when(i0 + 2 < cnt)
        def _():
            _wait_out(i0, xbuf0, so0)
            _in(i0 + 2, xbuf0, si0)

        @pl.when(i1 + 2 < cnt)
        def _():
            _wait_out(i1, xbuf1, so1)
            _in(i1 + 2, xbuf1, si1)

        return 0

    lax.fori_loop(0, NPAIR_A, pair_body, 0)
    _wait_out(0, xbuf0, so0)
    _wait_out(0, xbuf1, so1)


def _tc_stats_body(x_ref, lab_ref, seg_ref, ssq_ref):
    @pl.when(pl.program_id(0) == 0)
    def _():
        seg_ref[...] = jnp.zeros_like(seg_ref)
        ssq_ref[...] = jnp.zeros_like(ssq_ref)

    x = x_ref[...]
    labels = lab_ref[0, 0, :]
    onehot = (labels[:, None] == lax.broadcasted_iota(jnp.int32, (1, NB), 1)
              ).astype(jnp.float32)
    seg_ref[...] += lax.dot_general(
        onehot, x, (((0,), (0,)), ((), ())),
        preferred_element_type=jnp.float32)
    ssq_ref[...] += jnp.sum(x * x, axis=0, keepdims=True)


def _tc_apply_body(alias_ref, x_ref, lab_ref, tbl_ref, out_ref):
    del alias_ref
    labels = lab_ref[0, 0, :]
    onehot = (labels[:, None] == lax.broadcasted_iota(jnp.int32, (1, NB), 1)
              ).astype(jnp.float32)
    corr = lax.dot_general(
        onehot, tbl_ref[...], (((1,), (0,)), ((), ())),
        preferred_element_type=jnp.float32)
    out_ref[...] = x_ref[...] + corr


def _finalize_body(segp_ref, ssqp_ref, segtc_ref, ssqtc_ref, lab_ref,
                   tbl_ref):
    seg = jnp.sum(segp_ref[...], axis=0) + segtc_ref[...]
    ssq = jnp.sum(ssqp_ref[...], axis=0, keepdims=True) + ssqtc_ref[...]
    gm = jnp.sum(seg, axis=0, keepdims=True) / N
    gv = ssq / N - gm * gm
    inv_std = 1.0 / (jnp.sqrt(gv) + 1e-8)
    labs = lab_ref[:, 0, :]
    rows = []
    for b in range(NB):
        cnt = jnp.maximum(
            jnp.sum((labs == b).astype(jnp.float32)), 1.0)
        bm = seg[b:b + 1, :] / cnt
        rows.append(-(bm - gm) * inv_std)
    tbl_ref[...] = jnp.concatenate(rows, axis=1)


@jax.jit
def kernel(expression, batch_labels):
    mesh = plsc.VectorSubcoreMesh(core_axis_name="c", subcore_axis_name="s")

    stats = functools.partial(
        pl.kernel,
        mesh=mesh,
        out_type=[
            jax.ShapeDtypeStruct((NW, NB * G), jnp.float32),
            jax.ShapeDtypeStruct((NW, G), jnp.float32),
        ],
        scratch_types=[
            pltpu.VMEM((CHUNK, G), jnp.float32),
            pltpu.VMEM((CHUNK, G), jnp.float32),
            pltpu.VMEM((MAXC * CHUNK,), jnp.int32),
            pltpu.VMEM((NB * G,), jnp.float32),
            pltpu.VMEM((G,), jnp.float32),
            pltpu.SemaphoreType.DMA,
            pltpu.SemaphoreType.DMA,
        ],
    )(_sc_stats_body)
    seg_p, ssq_p = stats(expression, batch_labels)
    seg_p = seg_p.reshape(NW, NB, G)

    labels3 = batch_labels.reshape(N // TC_BLK, 1, TC_BLK)
    seg_tc, ssq_tc = pl.pallas_call(
        _tc_stats_body,
        grid=(TC_ROWS // TC_BLK,),
        in_specs=[
            pl.BlockSpec((TC_BLK, G), lambda i: (i, 0)),
            pl.BlockSpec((1, 1, TC_BLK), lambda i: (i, 0, 0)),
        ],
        out_specs=[
            pl.BlockSpec((NB, G), lambda i: (0, 0)),
            pl.BlockSpec((1, G), lambda i: (0, 0)),
        ],
        out_shape=[
            jax.ShapeDtypeStruct((NB, G), jnp.float32),
            jax.ShapeDtypeStruct((1, G), jnp.float32),
        ],
    )(expression, labels3)

    negtbl = pl.pallas_call(
        _finalize_body,
        out_shape=jax.ShapeDtypeStruct((1, NB * G), jnp.float32),
    )(seg_p, ssq_p, seg_tc, ssq_tc, labels3)

    apply_fn = functools.partial(
        pl.kernel,
        mesh=mesh,
        out_type=jax.ShapeDtypeStruct((N, G), jnp.float32),
        scratch_types=[
            pltpu.VMEM((CHUNK, G), jnp.float32),
            pltpu.VMEM((CHUNK, G), jnp.float32),
            pltpu.VMEM((MAXC_A * CHUNK,), jnp.int32),
            pltpu.VMEM((NB * G,), jnp.float32),
            pltpu.SemaphoreType.DMA,
            pltpu.SemaphoreType.DMA,
            pltpu.SemaphoreType.DMA,
            pltpu.SemaphoreType.DMA,
        ],
    )(_sc_apply_body)
    out_sc = apply_fn(expression, batch_labels, negtbl)

    negtbl2d = negtbl.reshape(NB, G)
    return pl.pallas_call(
        _tc_apply_body,
        grid=(TC_AROWS // TC_BLK,),
        in_specs=[
            pl.BlockSpec(memory_space=pl.ANY),
            pl.BlockSpec((TC_BLK, G), lambda i: (i, 0)),
            pl.BlockSpec((1, 1, TC_BLK), lambda i: (i, 0, 0)),
            pl.BlockSpec((NB, G), lambda i: (0, 0)),
        ],
        out_specs=pl.BlockSpec((TC_BLK, G), lambda i: (i, 0)),
        out_shape=jax.ShapeDtypeStruct((N, G), jnp.float32),
        input_output_aliases={0: 0},
    )(out_sc, expression, labels3, negtbl2d)
